# Initial kernel scaffold; baseline (speedup 1.0000x reference)
import jax
import jax.numpy as jnp
from jax.experimental import pallas as pl


def kernel(response, position_embed, response_embed):
    raise NotImplementedError("write your pallas kernel here")


# speedup vs baseline: 2.2735x; 2.2735x over previous
import functools

import jax
import jax.numpy as jnp
from jax import lax
from jax.experimental import pallas as pl
from jax.experimental.pallas import tpu as pltpu
from jax.experimental.pallas import tpu_sc as plsc

SEQ_LEN = 200
N_DIMS = 64
BATCH = 4096

NUM_CORES = 2
NUM_SUBCORES = 16
NUM_WORKERS = NUM_CORES * NUM_SUBCORES
ROWS_PER_WORKER = BATCH // NUM_WORKERS

GATHER_CHUNKS = ((0, 120), (120, 80))


def _body(resp_hbm, pos_hbm, tab_hbm, out_hbm, idx_v, pos_v, rows_v, sem):
    wid = lax.axis_index("s") * NUM_CORES + lax.axis_index("c")

    pltpu.sync_copy(pos_hbm, pos_v)

    def per_batch(i, carry):
        b = wid * ROWS_PER_WORKER + i
        pltpu.sync_copy(resp_hbm.at[b], idx_v)
        copies = []
        for off, n in GATHER_CHUNKS:
            copies.append(
                pltpu.async_copy(
                    tab_hbm.at[idx_v.at[pl.ds(off, n)]],
                    rows_v.at[pl.ds(off, n)],
                    sem,
                )
            )
        for cp in copies:
            cp.wait()

        def add_row(r, c):
            for l in range(N_DIMS // 16):
                sl = pl.ds(l * 16, 16)
                rows_v[r, sl] = rows_v[r, sl] + pos_v[r, sl]
            return c

        lax.fori_loop(0, SEQ_LEN, add_row, 0, unroll=2)
        pltpu.sync_copy(rows_v, out_hbm.at[b])
        return carry

    lax.fori_loop(0, ROWS_PER_WORKER, per_batch, 0)


@jax.jit
def _run(response, position_embed, response_embed):
    mesh = plsc.VectorSubcoreMesh(core_axis_name="c", subcore_axis_name="s")
    f = pl.kernel(
        _body,
        out_type=jax.ShapeDtypeStruct((BATCH, SEQ_LEN, N_DIMS), jnp.float32),
        mesh=mesh,
        scratch_types=[
            pltpu.VMEM((SEQ_LEN,), jnp.int32),
            pltpu.VMEM((SEQ_LEN, N_DIMS), jnp.float32),
            pltpu.VMEM((SEQ_LEN, N_DIMS), jnp.float32),
            pltpu.SemaphoreType.DMA,
        ],
        compiler_params=pltpu.CompilerParams(use_tc_tiling_on_sc=False),
    )
    return f(response, position_embed, response_embed)


def kernel(response, position_embed, response_embed):
    return _run(response.astype(jnp.int32), position_embed, response_embed)


# speedup vs baseline: 3.9877x; 1.7540x over previous
import jax
import jax.numpy as jnp
from jax import lax
from jax.experimental import pallas as pl
from jax.experimental.pallas import tpu as pltpu
from jax.experimental.pallas import tpu_sc as plsc

SEQ_LEN = 200
N_DIMS = 64
BATCH = 4096

NUM_CORES = 2
NUM_SUBCORES = 16
NUM_WORKERS = NUM_CORES * NUM_SUBCORES
ROWS_PER_WORKER = BATCH // NUM_WORKERS


def _body(resp_t_hbm, pos_hbm, tab_hbm, out_hbm, idx2, rows_a, rows_b, pos_v,
          sem_g, sem_o):
    wid = lax.axis_index("s") * NUM_CORES + lax.axis_index("c")
    b0 = wid * ROWS_PER_WORKER

    pltpu.sync_copy(pos_hbm, pos_v)

    pltpu.sync_copy(resp_t_hbm.at[0, pl.ds(b0, ROWS_PER_WORKER)], idx2.at[0])
    pltpu.async_copy(tab_hbm.at[idx2.at[0]], rows_a, sem_g)

    def out_slice(s):
        return out_hbm.at[pl.ds(b0, ROWS_PER_WORKER), pl.ds(s * N_DIMS, N_DIMS)]

    def step(i, carry):
        s0 = i * 2
        for k, (cur, oth) in ((0, (rows_a, rows_b)), (1, (rows_b, rows_a))):
            s = s0 + k
            pltpu.make_async_copy(tab_hbm.at[idx2.at[k]], cur, sem_g).wait()

            @pl.when(s >= 1)
            def _():
                pltpu.make_async_copy(oth, out_slice(s - 1), sem_o).wait()

            @pl.when(s <= SEQ_LEN - 2)
            def _():
                pltpu.sync_copy(
                    resp_t_hbm.at[s + 1, pl.ds(b0, ROWS_PER_WORKER)],
                    idx2.at[1 - k],
                )
                pltpu.async_copy(tab_hbm.at[idx2.at[1 - k]], oth, sem_g)

            p = [pos_v[s, pl.ds(16 * l, 16)] for l in range(N_DIMS // 16)]

            def add_row(r, c):
                for l in range(N_DIMS // 16):
                    sl = pl.ds(16 * l, 16)
                    cur[r, sl] = cur[r, sl] + p[l]
                return c

            lax.fori_loop(0, ROWS_PER_WORKER, add_row, 0, unroll=4)

            pltpu.async_copy(cur, out_slice(s), sem_o)
        return carry

    lax.fori_loop(0, SEQ_LEN // 2, step, 0)
    pltpu.make_async_copy(rows_b, out_slice(SEQ_LEN - 1), sem_o).wait()


@jax.jit
def _run(resp_t, position_embed, response_embed):
    mesh = plsc.VectorSubcoreMesh(core_axis_name="c", subcore_axis_name="s")
    f = pl.kernel(
        _body,
        out_type=jax.ShapeDtypeStruct((BATCH, SEQ_LEN * N_DIMS), jnp.float32),
        mesh=mesh,
        scratch_types=[
            pltpu.VMEM((2, ROWS_PER_WORKER), jnp.int32),
            pltpu.VMEM((ROWS_PER_WORKER, N_DIMS), jnp.float32),
            pltpu.VMEM((ROWS_PER_WORKER, N_DIMS), jnp.float32),
            pltpu.VMEM((SEQ_LEN, N_DIMS), jnp.float32),
            pltpu.SemaphoreType.DMA,
            pltpu.SemaphoreType.DMA,
        ],
        compiler_params=pltpu.CompilerParams(use_tc_tiling_on_sc=False),
    )
    out = f(resp_t, position_embed, response_embed)
    return out.reshape(BATCH, SEQ_LEN, N_DIMS)


def kernel(response, position_embed, response_embed):
    resp_t = response.astype(jnp.int32).T
    return _run(resp_t, position_embed, response_embed)


# speedup vs baseline: 5.2361x; 1.3131x over previous
import jax
import jax.numpy as jnp
from jax import lax
from jax.experimental import pallas as pl
from jax.experimental.pallas import tpu as pltpu
from jax.experimental.pallas import tpu_sc as plsc

SEQ_LEN = 200
N_DIMS = 64
BATCH = 4096

NUM_CORES = 2
NUM_SUBCORES = 16
NUM_WORKERS = NUM_CORES * NUM_SUBCORES
ROWS_PER_WORKER = BATCH // NUM_WORKERS

NBUF = 4
LOOKAHEAD = 2


def _body(resp_t_hbm, pos_hbm, tab_hbm, out_hbm, idx_all, rows, pos_v,
          sem_g, sem_o):
    wid = lax.axis_index("s") * NUM_CORES + lax.axis_index("c")
    b0 = wid * ROWS_PER_WORKER

    pltpu.sync_copy(pos_hbm, pos_v)
    pltpu.sync_copy(resp_t_hbm.at[:, pl.ds(b0, ROWS_PER_WORKER)], idx_all)

    def out_slice(s):
        return out_hbm.at[pl.ds(b0, ROWS_PER_WORKER), pl.ds(s * N_DIMS, N_DIMS)]

    def start_gather(s, k):
        pltpu.async_copy(tab_hbm.at[idx_all.at[s]], rows.at[k], sem_g.at[k])

    for s in range(LOOKAHEAD):
        start_gather(s, s)

    def step(i, carry):
        s0 = i * NBUF
        for k in range(NBUF):
            s = s0 + k
            cur = rows.at[k]
            pltpu.make_async_copy(
                tab_hbm.at[idx_all.at[s]], cur, sem_g.at[k]
            ).wait()

            s_pre = s + LOOKAHEAD - NBUF
            k_nxt = (k + LOOKAHEAD) % NBUF

            @pl.when(s_pre >= 0)
            def _():
                pltpu.make_async_copy(
                    rows.at[k_nxt], out_slice(s_pre), sem_o.at[k_nxt]
                ).wait()

            @pl.when(s + LOOKAHEAD <= SEQ_LEN - 1)
            def _():
                start_gather(s + LOOKAHEAD, k_nxt)

            p = [pos_v[s, pl.ds(16 * l, 16)] for l in range(N_DIMS // 16)]

            def add_row(r, c):
                for l in range(N_DIMS // 16):
                    sl = pl.ds(16 * l, 16)
                    cur[r, sl] = cur[r, sl] + p[l]
                return c

            lax.fori_loop(0, ROWS_PER_WORKER, add_row, 0, unroll=4)

            pltpu.async_copy(cur, out_slice(s), sem_o.at[k])
        return carry

    lax.fori_loop(0, SEQ_LEN // NBUF, step, 0)
    for s in range(SEQ_LEN - NBUF + LOOKAHEAD, SEQ_LEN):
        k = s % NBUF
        pltpu.make_async_copy(rows.at[k], out_slice(s), sem_o.at[k]).wait()


@jax.jit
def _run(resp_t, position_embed, response_embed):
    mesh = plsc.VectorSubcoreMesh(core_axis_name="c", subcore_axis_name="s")
    f = pl.kernel(
        _body,
        out_type=jax.ShapeDtypeStruct((BATCH, SEQ_LEN * N_DIMS), jnp.float32),
        mesh=mesh,
        scratch_types=[
            pltpu.VMEM((SEQ_LEN, ROWS_PER_WORKER), jnp.int32),
            pltpu.VMEM((NBUF, ROWS_PER_WORKER, N_DIMS), jnp.float32),
            pltpu.VMEM((SEQ_LEN, N_DIMS), jnp.float32),
            pltpu.SemaphoreType.DMA((NBUF,)),
            pltpu.SemaphoreType.DMA((NBUF,)),
        ],
        compiler_params=pltpu.CompilerParams(use_tc_tiling_on_sc=False),
    )
    out = f(resp_t, position_embed, response_embed)
    return out.reshape(BATCH, SEQ_LEN, N_DIMS)


def kernel(response, position_embed, response_embed):
    resp_t = response.astype(jnp.int32).T
    return _run(resp_t, position_embed, response_embed)
